# Initial kernel scaffold; baseline (speedup 1.0000x reference)
import jax
import jax.numpy as jnp
from jax.experimental import pallas as pl


def kernel(feat, W, b):
    raise NotImplementedError("write your pallas kernel here")


# speedup vs baseline: 2.6365x; 2.6365x over previous
import jax
import jax.numpy as jnp
from jax import lax
from jax.experimental import pallas as pl
from jax.experimental.pallas import tpu as pltpu

_B, _C, _N, _K = 8, 64, 2048, 16
_KEEP = _N // 2
_RT = 256
_NT = _N // _RT

_DN_T = (((0,), (0,)), ((), ()))
_DN_N = (((1,), (0,)), ((), ()))
_DN_RR = (((1,), (1,)), ((), ()))
_F32 = jnp.float32


def _dot(a, b, dn):
    return lax.dot_general(a, b, dn, preferred_element_type=_F32)


def _dotx(a, b, dn):
    return lax.dot_general(a, b, dn, precision=lax.Precision.HIGHEST,
                           preferred_element_type=_F32)


def _edge_pool_body(feat_ref, xx_ref, wpack_ref, out_ref, scol_ref, srow_ref):
    X = feat_ref[0]
    xx_row = xx_ref[0]
    w_row = wpack_ref[0:1, :]
    bias = wpack_ref[1:2, 0:1]

    lane_iota = lax.broadcasted_iota(jnp.int32, (1, _N), 1)
    I_rt = (lax.broadcasted_iota(jnp.int32, (_RT, _RT), 0) ==
            lax.broadcasted_iota(jnp.int32, (_RT, _RT), 1)).astype(_F32)
    neg_inf = _F32(-jnp.inf)

    for rt in range(_NT):
        sl = slice(rt * _RT, (rt + 1) * _RT)
        Xn = X[:, sl]
        G = _dot(Xn, X, _DN_T)
        inner = -2.0 * G
        xx_col = _dotx(I_rt, xx_row[:, sl], _DN_RR)
        D = ((-xx_col) - inner) - xx_row
        Dw = D
        smax = jnp.full((1, _RT), neg_inf, _F32)
        for _ in range(_K):
            m = jnp.max(Dw, axis=1, keepdims=True)
            jidx = jnp.min(jnp.where(Dw == m, lane_iota, _N),
                           axis=1, keepdims=True)
            oh = (lane_iota == jidx)
            Dw = jnp.where(oh, neg_inf, Dw)
            Xm = _dotx(X, oh.astype(_F32), _DN_RR)
            EF = jnp.concatenate([Xm - Xn, Xn], axis=0)
            s_i = _dot(w_row, EF, _DN_N) + bias
            smax = jnp.maximum(smax, s_i)
        srow_ref[0:1, sl] = smax
        scol_ref[sl, :] = _dotx(I_rt, smax, _DN_RR)

    s_row = srow_ref[0:1, :]
    r_vals = lax.broadcasted_iota(jnp.int32, (1, _KEEP), 1)
    gathered = jnp.zeros((_C, _KEEP), _F32)
    tval = jnp.zeros((1, _KEEP), _F32)
    for rt in range(_NT):
        sl = slice(rt * _RT, (rt + 1) * _RT)
        s_col = scol_ref[sl, :]
        n_iota = lax.broadcasted_iota(jnp.int32, (_RT, 1), 0) + rt * _RT
        gt = (s_row > s_col) | ((s_row == s_col) & (lane_iota < n_iota))
        rank = jnp.sum(gt.astype(jnp.int32), axis=1, keepdims=True)
        P = (rank == r_vals).astype(_F32)
        gathered = gathered + _dotx(X[:, sl], P, _DN_N)
        tval = tval + _dotx(s_row[:, sl], P, _DN_N)
    out_ref[0] = gathered * jnp.tanh(tval)


def kernel(feat, W, b):
    xx = jnp.sum(feat * feat, axis=1).reshape(_B, 1, _N)
    wpack = jnp.zeros((8, 2 * _C), _F32)
    wpack = wpack.at[0, :].set(W[0, :, 0, 0])
    wpack = wpack.at[1, 0].set(b[0])
    return pl.pallas_call(
        _edge_pool_body,
        grid=(_B,),
        in_specs=[
            pl.BlockSpec((1, _C, _N), lambda i: (i, 0, 0)),
            pl.BlockSpec((1, 1, _N), lambda i: (i, 0, 0)),
            pl.BlockSpec((8, 2 * _C), lambda i: (0, 0)),
        ],
        out_specs=pl.BlockSpec((1, _C, _KEEP), lambda i: (i, 0, 0)),
        out_shape=jax.ShapeDtypeStruct((_B, _C, _KEEP), _F32),
        scratch_shapes=[
            pltpu.VMEM((_N, 1), _F32),
            pltpu.VMEM((1, _N), _F32),
        ],
    )(feat, xx, wpack)


# speedup vs baseline: 5.8259x; 2.2097x over previous
import jax
import jax.numpy as jnp
from jax import lax
from jax.experimental import pallas as pl
from jax.experimental.pallas import tpu as pltpu

_B, _C, _N, _K = 8, 64, 2048, 16
_KEEP = _N // 2
_RT = 256
_NT = _N // _RT
_TOPP = 4

_DN_T = (((0,), (0,)), ((), ()))
_DN_N = (((1,), (0,)), ((), ()))
_DN_RR = (((1,), (1,)), ((), ()))
_F32 = jnp.float32


def _dot(a, b, dn):
    return lax.dot_general(a, b, dn, preferred_element_type=_F32)


def _dotx(a, b, dn):
    return lax.dot_general(a, b, dn, precision=lax.Precision.HIGHEST,
                           preferred_element_type=_F32)


def _edge_pool_body(feat_ref, featT_ref, xx_ref, xxT_ref, wpack_ref, out_ref,
                    scol_ref, srow_ref):
    X = feat_ref[0]
    XT = featT_ref[0]
    xx_row = xx_ref[0]
    xx_colf = xxT_ref[0]
    w_row = wpack_ref[0:1, :]
    bias = wpack_ref[1:2, 0:1]

    lane_iota = lax.broadcasted_iota(jnp.int32, (1, _N), 1)
    k_iota = lax.broadcasted_iota(jnp.int32, (1, _K), 1)
    I_rt = (lax.broadcasted_iota(jnp.int32, (_RT, _RT), 0) ==
            lax.broadcasted_iota(jnp.int32, (_RT, _RT), 1)).astype(_F32)
    I_2c = (lax.broadcasted_iota(jnp.int32, (2 * _C, 2 * _C), 0) ==
            lax.broadcasted_iota(jnp.int32, (2 * _C, 2 * _C), 1)).astype(_F32)
    neg_inf = _F32(-jnp.inf)

    w_col = _dotx(I_2c, w_row, _DN_RR)
    p_row = _dotx(w_row[:, 0:_C], X, _DN_N)

    for rt in range(_NT):
        sl = slice(rt * _RT, (rt + 1) * _RT)
        Xn = X[:, sl]
        G = _dot(Xn, X, _DN_T)
        inner = -2.0 * G
        xx_col = xx_colf[sl, :]
        D = ((-xx_col) - inner) - xx_row
        Dw = D
        jlist, plist = [], []
        for _ in range(_K):
            m = jnp.max(Dw, axis=1, keepdims=True)
            jidx = jnp.min(jnp.where(Dw == m, lane_iota, _N),
                           axis=1, keepdims=True)
            oh = (lane_iota == jidx)
            Dw = jnp.where(oh, neg_inf, Dw)
            plist.append(jnp.max(jnp.where(oh, p_row, neg_inf),
                                 axis=1, keepdims=True))
            jlist.append(jidx)
        Jmat = jnp.concatenate(jlist, axis=1)
        Pmat = jnp.concatenate(plist, axis=1)

        XnT = XT[sl, :]
        smax = jnp.full((_RT, 1), neg_inf, _F32)
        Pw = Pmat
        for _ in range(_TOPP):
            pm = jnp.max(Pw, axis=1, keepdims=True)
            kidx = jnp.min(jnp.where(Pw == pm, k_iota, _K),
                           axis=1, keepdims=True)
            ohk = (k_iota == kidx)
            Pw = jnp.where(ohk, neg_inf, Pw)
            jc = jnp.sum(jnp.where(ohk, Jmat, 0), axis=1, keepdims=True)
            ohc = (lane_iota == jc).astype(_F32)
            Xm = _dotx(ohc, XT, _DN_N)
            EF = jnp.concatenate([Xm - XnT, XnT], axis=1)
            s_t = _dot(EF, w_col, _DN_N) + bias
            smax = jnp.maximum(smax, s_t)
        scol_ref[sl, :] = smax
        srow_ref[0:1, sl] = _dotx(smax, I_rt, _DN_T)

    s_row = srow_ref[0:1, :]
    r_vals = lax.broadcasted_iota(jnp.int32, (1, _KEEP), 1)
    gathered = jnp.zeros((_C, _KEEP), _F32)
    tval = jnp.zeros((1, _KEEP), _F32)
    for rt in range(_NT):
        sl = slice(rt * _RT, (rt + 1) * _RT)
        s_col = scol_ref[sl, :]
        n_iota = lax.broadcasted_iota(jnp.int32, (_RT, 1), 0) + rt * _RT
        gt = (s_row > s_col) | ((s_row == s_col) & (lane_iota < n_iota))
        rank = jnp.sum(gt.astype(jnp.int32), axis=1, keepdims=True)
        P = (rank == r_vals).astype(_F32)
        gathered = gathered + _dotx(X[:, sl], P, _DN_N)
        tval = tval + _dotx(s_row[:, sl], P, _DN_N)
    out_ref[0] = gathered * jnp.tanh(tval)


def kernel(feat, W, b):
    featT = jnp.transpose(feat, (0, 2, 1))
    xx = jnp.sum(feat * feat, axis=1)
    xx3 = xx.reshape(_B, 1, _N)
    xxT = xx.reshape(_B, _N, 1)
    wpack = jnp.zeros((8, 2 * _C), _F32)
    wpack = wpack.at[0, :].set(W[0, :, 0, 0])
    wpack = wpack.at[1, 0].set(b[0])
    return pl.pallas_call(
        _edge_pool_body,
        grid=(_B,),
        in_specs=[
            pl.BlockSpec((1, _C, _N), lambda i: (i, 0, 0)),
            pl.BlockSpec((1, _N, _C), lambda i: (i, 0, 0)),
            pl.BlockSpec((1, 1, _N), lambda i: (i, 0, 0)),
            pl.BlockSpec((1, _N, 1), lambda i: (i, 0, 0)),
            pl.BlockSpec((8, 2 * _C), lambda i: (0, 0)),
        ],
        out_specs=pl.BlockSpec((1, _C, _KEEP), lambda i: (i, 0, 0)),
        out_shape=jax.ShapeDtypeStruct((_B, _C, _KEEP), _F32),
        scratch_shapes=[
            pltpu.VMEM((_N, 1), _F32),
            pltpu.VMEM((1, _N), _F32),
        ],
    )(feat, featT, xx3, xxT, wpack)
